# baseline (speedup 1.0000x reference)
import functools

import jax
import jax.numpy as jnp
from jax import lax
from jax.experimental import pallas as pl
from jax.experimental.pallas import tpu as pltpu
from jax.experimental.pallas import tpu_sc as plsc

B = 16384
NCAT = 26
CATSZ = 100
NCONT = 13
D = NCAT * CATSZ + NCONT


_RBLK = 256


def _dense_body(xc_ref, med_ref, fac_ref, out_ref):
    x = xc_ref[...]
    xs = fac_ref[...] * (x - med_ref[...])
    y = xs / jnp.sqrt(1.0 + (xs * (1.0 / 3.0)) ** 2)
    out_ref[...] = jnp.zeros((_RBLK, D), jnp.float32)
    out_ref[:, NCAT * CATSZ:D] = y


_dense_call = pl.pallas_call(
    _dense_body,
    grid=(B // _RBLK,),
    in_specs=[
        pl.BlockSpec((_RBLK, NCONT), lambda i: (i, 0)),
        pl.BlockSpec((1, NCONT), lambda i: (0, 0)),
        pl.BlockSpec((1, NCONT), lambda i: (0, 0)),
    ],
    out_specs=pl.BlockSpec((_RBLK, D), lambda i: (i, 0)),
    out_shape=jax.ShapeDtypeStruct((B, D), jnp.float32),
)


_NW = 32
_RPW = B // _NW
_WPW = _RPW * NCAT
_IDXROWS = _WPW // 128

_sc_mesh = plsc.VectorSubcoreMesh(core_axis_name="c", subcore_axis_name="s")


@functools.partial(
    pl.kernel,
    mesh=_sc_mesh,
    scratch_types=[
        pltpu.VMEM((_WPW,), jnp.int32),
        pltpu.VMEM((_WPW,), jnp.int32),
        pltpu.VMEM((_WPW,), jnp.float32),
        pltpu.SemaphoreType.DMA,
    ],
)
def _sc_scatter(out_hbm, cat_hbm, cat_v, idx_v, ones_v, sem):
    wid = lax.axis_index("s") * 2 + lax.axis_index("c")
    row0 = wid * _RPW
    pltpu.sync_copy(cat_hbm.at[pl.ds(row0 * NCAT, _WPW)], cat_v)

    lane = lax.iota(jnp.int32, 16)
    lane100 = lane * CATSZ
    one16 = jnp.full((16,), 1.0, jnp.float32)

    def fill(g, qr):
        q, r = qr
        for b in range(8):
            cat = cat_v[pl.ds(g * 128 + b * 16, 16)]
            c13 = jnp.where(lane >= 26 - r, NCONT, 0)
            scalar = (row0 + q) * D + r * CATSZ
            idx_v[pl.ds(g * 128 + b * 16, 16)] = lane100 + c13 + cat + scalar
            ones_v[pl.ds(g * 128 + b * 16, 16)] = one16
            wrap = r + 16 >= 26
            q = q + jnp.where(wrap, 1, 0)
            r = r + jnp.where(wrap, 16 - 26, 16)
        return (q, r)

    lax.fori_loop(0, _IDXROWS, fill, (jnp.int32(0), jnp.int32(0)))

    pltpu.async_copy(ones_v, out_hbm.at[idx_v], sem).wait()


def kernel(x_cat, x_cont, median, factors):
    cat_flat = x_cat.astype(jnp.int32).reshape(-1)
    dense = _dense_call(
        x_cont.astype(jnp.float32),
        median.astype(jnp.float32).reshape(1, NCONT),
        factors.astype(jnp.float32).reshape(1, NCONT),
    )
    out_ref = jax.new_ref(dense.reshape(-1))
    _sc_scatter(out_ref, cat_flat)
    return out_ref[...].reshape(B, D)


# speedup vs baseline: 1.2285x; 1.2285x over previous
import functools

import jax
import jax.numpy as jnp
from jax import lax
from jax.experimental import pallas as pl
from jax.experimental.pallas import tpu as pltpu
from jax.experimental.pallas import tpu_sc as plsc

B = 16384
NCAT = 26
CATSZ = 100
NCONT = 13
D = NCAT * CATSZ + NCONT


_RBLK = 256
P = 2688


def _finish_body(pad_ref, xc_ref, med_ref, fac_ref, out_ref):
    x = xc_ref[...]
    xs = fac_ref[...] * (x - med_ref[...])
    y = xs / jnp.sqrt(1.0 + (xs * (1.0 / 3.0)) ** 2)
    out_ref[...] = pad_ref[:, 0:D]
    out_ref[:, NCAT * CATSZ:D] = y


_finish_call = pl.pallas_call(
    _finish_body,
    grid=(B // _RBLK,),
    in_specs=[
        pl.BlockSpec((_RBLK, P), lambda i: (i, 0)),
        pl.BlockSpec((_RBLK, NCONT), lambda i: (i, 0)),
        pl.BlockSpec((1, NCONT), lambda i: (0, 0)),
        pl.BlockSpec((1, NCONT), lambda i: (0, 0)),
    ],
    out_specs=pl.BlockSpec((_RBLK, D), lambda i: (i, 0)),
    out_shape=jax.ShapeDtypeStruct((B, D), jnp.float32),
)


_NW = 32
_RPW = B // _NW
_WPW = _RPW * NCAT
_IDXROWS = _WPW // 128

_sc_mesh = plsc.VectorSubcoreMesh(core_axis_name="c", subcore_axis_name="s")


@functools.partial(
    pl.kernel,
    mesh=_sc_mesh,
    scratch_types=[
        pltpu.VMEM((_WPW,), jnp.int32),
        pltpu.VMEM((_WPW,), jnp.int32),
        pltpu.VMEM((_WPW,), jnp.float32),
        pltpu.SemaphoreType.DMA,
    ],
)
def _sc_scatter(out_hbm, cat_hbm, cat_v, idx_v, ones_v, sem):
    wid = lax.axis_index("s") * 2 + lax.axis_index("c")
    row0 = wid * _RPW
    pltpu.sync_copy(cat_hbm.at[pl.ds(row0 * NCAT, _WPW)], cat_v)

    lane = lax.iota(jnp.int32, 16)
    lane100 = lane * CATSZ
    one16 = jnp.full((16,), 1.0, jnp.float32)

    def fill(g, qr):
        q, r = qr
        for b in range(8):
            cat = cat_v[pl.ds(g * 128 + b * 16, 16)]
            c13 = jnp.where(lane >= 26 - r, P - NCAT * CATSZ, 0)
            scalar = (row0 + q) * P + r * CATSZ
            idx_v[pl.ds(g * 128 + b * 16, 16)] = lane100 + c13 + cat + scalar
            ones_v[pl.ds(g * 128 + b * 16, 16)] = one16
            wrap = r + 16 >= 26
            q = q + jnp.where(wrap, 1, 0)
            r = r + jnp.where(wrap, 16 - 26, 16)
        return (q, r)

    lax.fori_loop(0, _IDXROWS, fill, (jnp.int32(0), jnp.int32(0)))

    pltpu.async_copy(ones_v, out_hbm.at[idx_v], sem).wait()


def kernel(x_cat, x_cont, median, factors):
    cat_flat = x_cat.astype(jnp.int32).reshape(-1)
    pad_ref = jax.new_ref(jnp.zeros((B * P,), jnp.float32))
    _sc_scatter(pad_ref, cat_flat)
    padded = pad_ref[...].reshape(B, P)
    return _finish_call(
        padded,
        x_cont.astype(jnp.float32),
        median.astype(jnp.float32).reshape(1, NCONT),
        factors.astype(jnp.float32).reshape(1, NCONT),
    )


# speedup vs baseline: 1.4632x; 1.1911x over previous
import functools

import jax
import jax.numpy as jnp
from jax import lax
from jax.experimental import pallas as pl
from jax.experimental.pallas import tpu as pltpu
from jax.experimental.pallas import tpu_sc as plsc
from jax._src.pallas import mpmd as _mpmd

B = 16384
NCAT = 26
CATSZ = 100
NCONT = 13
D = NCAT * CATSZ + NCONT
PT = 2616


_CBLK = 256


def _finish_body(pad_ref, xc_ref, med_ref, fac_ref, out_ref):
    xT = xc_ref[...]
    xs = fac_ref[...] * (xT - med_ref[...])
    y = xs / jnp.sqrt(1.0 + (xs * (1.0 / 3.0)) ** 2)
    out_ref[...] = pad_ref[0:D, :]
    out_ref[NCAT * CATSZ:D, :] = y


_finish_call = pl.pallas_call(
    _finish_body,
    grid=(B // _CBLK,),
    in_specs=[
        pl.BlockSpec((PT, _CBLK), lambda i: (0, i)),
        pl.BlockSpec((NCONT, _CBLK), lambda i: (0, i)),
        pl.BlockSpec((NCONT, 1), lambda i: (0, 0)),
        pl.BlockSpec((NCONT, 1), lambda i: (0, 0)),
    ],
    out_specs=pl.BlockSpec((D, _CBLK), lambda i: (0, i)),
    out_shape=jax.ShapeDtypeStruct((D, B), jnp.float32),
)


_NW = 32
_RPW = B // _NW
_WPW = _RPW * NCAT
_IDXROWS = _WPW // 128

_sc_mesh = plsc.VectorSubcoreMesh(core_axis_name="c", subcore_axis_name="s")


def _sc_body(zin_hbm, cat_hbm, out_hbm, cat_v, idx_v, ones_v, sem):
    del zin_hbm
    wid = lax.axis_index("s") * 2 + lax.axis_index("c")
    row0 = wid * _RPW
    pltpu.sync_copy(cat_hbm.at[pl.ds(row0 * NCAT, _WPW)], cat_v)

    lane = lax.iota(jnp.int32, 16)
    lane_cb = lane * (CATSZ * B)
    one16 = jnp.full((16,), 1.0, jnp.float32)
    c_coeff = 1 - NCAT * CATSZ * B

    def fill(g, qr):
        q, r = qr
        for b in range(8):
            cat = cat_v[pl.ds(g * 128 + b * 16, 16)]
            cterm = jnp.where(lane >= 26 - r, c_coeff, 0)
            scalar = CATSZ * B * r + row0 + q
            idx_v[pl.ds(g * 128 + b * 16, 16)] = lane_cb + cat * B + cterm + scalar
            ones_v[pl.ds(g * 128 + b * 16, 16)] = one16
            wrap = r + 16 >= 26
            q = q + jnp.where(wrap, 1, 0)
            r = r + jnp.where(wrap, 16 - 26, 16)
        return (q, r)

    lax.fori_loop(0, _IDXROWS, fill, (jnp.int32(0), jnp.int32(0)))

    pltpu.async_copy(ones_v, out_hbm.at[idx_v], sem).wait()


_sc_scatter = _mpmd._mpmd_map(
    [(_sc_mesh, _sc_body)],
    out_types=jax.ShapeDtypeStruct((PT * B,), jnp.float32),
    input_output_aliases={0: 0},
    scratch_types=[
        pltpu.VMEM((_WPW,), jnp.int32),
        pltpu.VMEM((_WPW,), jnp.int32),
        pltpu.VMEM((_WPW,), jnp.float32),
        pltpu.SemaphoreType.DMA,
    ],
)


def kernel(x_cat, x_cont, median, factors):
    cat_flat = x_cat.astype(jnp.int32).reshape(-1)
    zeros_flat = jnp.zeros((PT * B,), jnp.float32)
    pad_flat = _sc_scatter(zeros_flat, cat_flat)
    out_t = _finish_call(
        pad_flat.reshape(PT, B),
        x_cont.astype(jnp.float32).T,
        median.astype(jnp.float32).reshape(NCONT, 1),
        factors.astype(jnp.float32).reshape(NCONT, 1),
    )
    return out_t.T


# speedup vs baseline: 1.4660x; 1.0019x over previous
import functools

import jax
import jax.numpy as jnp
from jax import lax
from jax.experimental import pallas as pl
from jax.experimental.pallas import tpu as pltpu
from jax.experimental.pallas import tpu_sc as plsc
from jax._src.pallas import mpmd as _mpmd

B = 16384
NCAT = 26
CATSZ = 100
NCONT = 13
D = NCAT * CATSZ + NCONT
PT = 2616


_CBLK = 256


def _finish_body(pad_ref, xt_ref, med_ref, fac_ref, out_ref):
    i = pl.program_id(0)
    v = pad_ref[...].reshape(8, B)
    xt = xt_ref[...].reshape(8, B)
    med = med_ref[...].reshape(8, B)
    fac = fac_ref[...].reshape(8, B)
    xs = fac * (xt - med)
    y = xs / jnp.sqrt(1.0 + (xs * (1.0 / 3.0)) ** 2)
    row = i * 8 + lax.broadcasted_iota(jnp.int32, (8, B), 0)
    out_ref[...] = jnp.where(
        (row >= NCAT * CATSZ) & (row < D), y, v
    )


_finish_call = pl.pallas_call(
    _finish_body,
    grid=(PT // 8,),
    in_specs=[
        pl.BlockSpec((8 * B,), lambda i: (i,)),
        pl.BlockSpec((8 * B,), lambda i: jnp.maximum(i - 325, 0)),
        pl.BlockSpec((8 * B,), lambda i: jnp.maximum(i - 325, 0)),
        pl.BlockSpec((8 * B,), lambda i: jnp.maximum(i - 325, 0)),
    ],
    out_specs=pl.BlockSpec((8, B), lambda i: (i, 0)),
    out_shape=jax.ShapeDtypeStruct((D, B), jnp.float32),
)


_NW = 32
_RPW = B // _NW
_WPW = _RPW * NCAT
_IDXROWS = _WPW // 128

_sc_mesh = plsc.VectorSubcoreMesh(core_axis_name="c", subcore_axis_name="s")


def _sc_body(zin_hbm, cat_hbm, out_hbm, cat_v, idx_v, ones_v, sem):
    del zin_hbm
    wid = lax.axis_index("s") * 2 + lax.axis_index("c")
    row0 = wid * _RPW
    pltpu.sync_copy(cat_hbm.at[pl.ds(row0 * NCAT, _WPW)], cat_v)

    lane = lax.iota(jnp.int32, 16)
    lane_cb = lane * (CATSZ * B)
    one16 = jnp.full((16,), 1.0, jnp.float32)
    c_coeff = 1 - NCAT * CATSZ * B

    def fill(g, qr):
        q, r = qr
        for b in range(8):
            cat = cat_v[pl.ds(g * 128 + b * 16, 16)]
            cterm = jnp.where(lane >= 26 - r, c_coeff, 0)
            scalar = CATSZ * B * r + row0 + q
            idx_v[pl.ds(g * 128 + b * 16, 16)] = lane_cb + cat * B + cterm + scalar
            ones_v[pl.ds(g * 128 + b * 16, 16)] = one16
            wrap = r + 16 >= 26
            q = q + jnp.where(wrap, 1, 0)
            r = r + jnp.where(wrap, 16 - 26, 16)
        return (q, r)

    lax.fori_loop(0, _IDXROWS, fill, (jnp.int32(0), jnp.int32(0)))

    pltpu.async_copy(ones_v, out_hbm.at[idx_v], sem).wait()


_sc_scatter = _mpmd._mpmd_map(
    [(_sc_mesh, _sc_body)],
    out_types=jax.ShapeDtypeStruct((PT * B,), jnp.float32),
    input_output_aliases={0: 0},
    scratch_types=[
        pltpu.VMEM((_WPW,), jnp.int32),
        pltpu.VMEM((_WPW,), jnp.int32),
        pltpu.VMEM((_WPW,), jnp.float32),
        pltpu.SemaphoreType.DMA,
    ],
)


def kernel(x_cat, x_cont, median, factors):
    cat_flat = x_cat.astype(jnp.int32).reshape(-1)
    zeros_flat = jnp.zeros((PT * B,), jnp.float32)
    pad_flat = _sc_scatter(zeros_flat, cat_flat)
    xt_flat = x_cont.astype(jnp.float32).T.reshape(-1)
    med_flat = jnp.broadcast_to(
        median.astype(jnp.float32)[:, None], (NCONT, B)
    ).reshape(-1)
    fac_flat = jnp.broadcast_to(
        factors.astype(jnp.float32)[:, None], (NCONT, B)
    ).reshape(-1)
    out_t = _finish_call(pad_flat, xt_flat, med_flat, fac_flat)
    return out_t.T


# speedup vs baseline: 1.4666x; 1.0004x over previous
import jax
import jax.numpy as jnp
from jax import lax
from jax.experimental import pallas as pl
from jax.experimental.pallas import tpu as pltpu
from jax.experimental.pallas import tpu_sc as plsc
from jax._src.pallas import mpmd as _mpmd

B = 16384
NCAT = 26
CATSZ = 100
NCONT = 13
D = NCAT * CATSZ + NCONT
PT = 2616


def _finish_body(pad_ref, xt_ref, med_ref, fac_ref, out_ref):
    i = pl.program_id(0)
    v = pad_ref[...].reshape(8, B)
    xt = xt_ref[...].reshape(8, B)
    med = med_ref[...].reshape(8, B)
    fac = fac_ref[...].reshape(8, B)
    xs = fac * (xt - med)
    y = xs / jnp.sqrt(1.0 + (xs * (1.0 / 3.0)) ** 2)
    row = i * 8 + lax.broadcasted_iota(jnp.int32, (8, B), 0)
    out_ref[...] = jnp.where(
        (row >= NCAT * CATSZ) & (row < D), y, v
    )


_finish_call = pl.pallas_call(
    _finish_body,
    grid=(PT // 8,),
    in_specs=[
        pl.BlockSpec((8 * B,), lambda i: (i,)),
        pl.BlockSpec((8 * B,), lambda i: jnp.maximum(i - 325, 0)),
        pl.BlockSpec((8 * B,), lambda i: jnp.maximum(i - 325, 0)),
        pl.BlockSpec((8 * B,), lambda i: jnp.maximum(i - 325, 0)),
    ],
    out_specs=pl.BlockSpec((8, B), lambda i: (i, 0)),
    out_shape=jax.ShapeDtypeStruct((D, B), jnp.float32),
)


_NW = 32
_RPW = B // _NW
_WPW = _RPW * NCAT
_IDXROWS = _WPW // 128

_sc_mesh = plsc.VectorSubcoreMesh(core_axis_name="c", subcore_axis_name="s")


def _sc_body(zin_hbm, cat_hbm, out_hbm, cat_v, idx_v, ones_v, sem):
    del zin_hbm
    wid = lax.axis_index("s") * 2 + lax.axis_index("c")
    row0 = wid * _RPW
    pltpu.sync_copy(cat_hbm.at[pl.ds(row0 * NCAT, _WPW)], cat_v)

    lane = lax.iota(jnp.int32, 16)
    lane_cb = lane * (CATSZ * B)
    one16 = jnp.full((16,), 1.0, jnp.float32)
    c_coeff = 1 - NCAT * CATSZ * B

    def fill(g, qr):
        q, r = qr
        for b in range(8):
            cat = cat_v[pl.ds(g * 128 + b * 16, 16)]
            cterm = jnp.where(lane >= 26 - r, c_coeff, 0)
            scalar = CATSZ * B * r + row0 + q
            idx_v[pl.ds(g * 128 + b * 16, 16)] = lane_cb + cat * B + cterm + scalar
            ones_v[pl.ds(g * 128 + b * 16, 16)] = one16
            wrap = r + 16 >= 26
            q = q + jnp.where(wrap, 1, 0)
            r = r + jnp.where(wrap, 16 - 26, 16)
        return (q, r)

    lax.fori_loop(0, _IDXROWS, fill, (jnp.int32(0), jnp.int32(0)))

    pltpu.async_copy(ones_v, out_hbm.at[idx_v], sem).wait()


_sc_scatter = _mpmd._mpmd_map(
    [(_sc_mesh, _sc_body)],
    out_types=jax.ShapeDtypeStruct((PT * B,), jnp.float32),
    input_output_aliases={0: 0},
    scratch_types=[
        pltpu.VMEM((_WPW,), jnp.int32),
        pltpu.VMEM((_WPW,), jnp.int32),
        pltpu.VMEM((_WPW,), jnp.float32),
        pltpu.SemaphoreType.DMA,
    ],
)


def kernel(x_cat, x_cont, median, factors):
    cat_flat = x_cat.astype(jnp.int32).reshape(-1)
    zeros_flat = jnp.zeros((PT * B,), jnp.float32)
    pad_flat = _sc_scatter(zeros_flat, cat_flat)
    xt_flat = x_cont.astype(jnp.float32).T.reshape(-1)
    med_flat = jnp.broadcast_to(
        median.astype(jnp.float32)[:, None], (NCONT, B)
    ).reshape(-1)
    fac_flat = jnp.broadcast_to(
        factors.astype(jnp.float32)[:, None], (NCONT, B)
    ).reshape(-1)
    out_t = _finish_call(pad_flat, xt_flat, med_flat, fac_flat)
    return out_t.T
